# baseline (device time: 10322 ns/iter reference)
import jax
import jax.numpy as jnp
from jax import lax
from jax.experimental import pallas as pl
from jax.experimental.pallas import tpu as pltpu

N_DEV = 4


def kernel(x, router_W, route_idx, expert_W):
    n, d = x.shape
    e_per, _, h = expert_W.shape

    xb = x.astype(jnp.bfloat16)
    ws = expert_W.astype(jnp.bfloat16).reshape(e_per * d, h)

    def body(x_ref, idx_ref, ew_ref, out_ref, comm_ref, outv,
             send_sems, recv_sems, out_sem):
        my_i = lax.axis_index("i")
        left = (my_i + N_DEV - 1) % N_DEV
        right = (my_i + 1) % N_DEV
        diag = (my_i + 2) % N_DEV

        barrier_sem = pltpu.get_barrier_semaphore()
        for nbr in (right, left, diag):
            pl.semaphore_signal(
                barrier_sem, inc=1,
                device_id=(nbr,), device_id_type=pl.DeviceIdType.MESH,
            )

        xv = x_ref[:, :]
        idx = idx_ref[:, :]
        e0 = my_i * e_per
        xm = jnp.concatenate(
            [jnp.where(idx == e0 + k, xv, jnp.bfloat16(0.0))
             for k in range(e_per)],
            axis=1,
        )
        acc = jnp.dot(xm, ew_ref[:, :], preferred_element_type=jnp.float32)
        comm_ref[0, :, :] = acc.astype(jnp.bfloat16)

        pl.semaphore_wait(barrier_sem, 3)

        rdmas = []
        for s, tgt in ((1, right), (2, left), (3, diag)):
            rdma = pltpu.make_async_remote_copy(
                src_ref=comm_ref.at[0],
                dst_ref=comm_ref.at[s],
                send_sem=send_sems.at[s - 1],
                recv_sem=recv_sems.at[s - 1],
                device_id=(tgt,),
                device_id_type=pl.DeviceIdType.MESH,
            )
            rdma.start()
            rdmas.append(rdma)

        rdmas[0].wait_recv()
        rdmas[1].wait_recv()
        outv[:, :] = (comm_ref[0, :, :] + comm_ref[1, :, :]) + comm_ref[2, :, :]
        rdmas[2].wait_recv()
        outv[:, :] += comm_ref[3, :, :]
        out_copy = pltpu.make_async_copy(outv, out_ref, out_sem)
        out_copy.start()

        for rdma in rdmas:
            rdma.wait_send()
        out_copy.wait()

    return pl.pallas_call(
        body,
        out_shape=jax.ShapeDtypeStruct((n, h), jnp.bfloat16),
        in_specs=[pl.BlockSpec(memory_space=pltpu.VMEM)] * 3,
        out_specs=pl.BlockSpec(memory_space=pl.ANY),
        scratch_shapes=[
            pltpu.VMEM((N_DEV, n, h), jnp.bfloat16),
            pltpu.VMEM((n, h), jnp.bfloat16),
            pltpu.SemaphoreType.DMA((3,)),
            pltpu.SemaphoreType.DMA((3,)),
            pltpu.SemaphoreType.DMA,
        ],
        compiler_params=pltpu.CompilerParams(collective_id=0),
    )(xb, route_idx, ws)


# device time: 10131 ns/iter; 1.0189x vs baseline; 1.0189x over previous
import jax
import jax.numpy as jnp
from jax import lax
from jax.experimental import pallas as pl
from jax.experimental.pallas import tpu as pltpu

N_DEV = 4


def kernel(x, router_W, route_idx, expert_W):
    n, d = x.shape
    e_per, _, h = expert_W.shape

    xb = x.astype(jnp.bfloat16)
    ws = expert_W.astype(jnp.bfloat16).reshape(e_per * d, h)

    def body(x_ref, idx_ref, ew_ref, out_ref, comm_ref, send_sems, recv_sems):
        my_i = lax.axis_index("i")
        left = (my_i + N_DEV - 1) % N_DEV
        right = (my_i + 1) % N_DEV
        diag = (my_i + 2) % N_DEV

        barrier_sem = pltpu.get_barrier_semaphore()
        for nbr in (right, left, diag):
            pl.semaphore_signal(
                barrier_sem, inc=1,
                device_id=(nbr,), device_id_type=pl.DeviceIdType.MESH,
            )

        xv = x_ref[:, :]
        idx = idx_ref[:, :]
        e0 = my_i * e_per
        xm = jnp.concatenate(
            [jnp.where(idx == e0 + k, xv, jnp.bfloat16(0.0))
             for k in range(e_per)],
            axis=1,
        )
        acc = jnp.dot(xm, ew_ref[:, :], preferred_element_type=jnp.float32)
        comm_ref[0, :, :] = acc.astype(jnp.bfloat16)

        pl.semaphore_wait(barrier_sem, 3)

        rdmas = []
        for s, tgt in ((1, right), (2, left), (3, diag)):
            rdma = pltpu.make_async_remote_copy(
                src_ref=comm_ref.at[0],
                dst_ref=comm_ref.at[s],
                send_sem=send_sems.at[s - 1],
                recv_sem=recv_sems.at[s - 1],
                device_id=(tgt,),
                device_id_type=pl.DeviceIdType.MESH,
            )
            rdma.start()
            rdmas.append(rdma)

        rdmas[0].wait_recv()
        rdmas[1].wait_recv()
        partial = (comm_ref[0, :, :] + comm_ref[1, :, :]) + comm_ref[2, :, :]
        rdmas[2].wait_recv()
        out_ref[:, :] = partial + comm_ref[3, :, :]

        for rdma in rdmas:
            rdma.wait_send()

    return pl.pallas_call(
        body,
        out_shape=jax.ShapeDtypeStruct((n, h), jnp.bfloat16),
        in_specs=[pl.BlockSpec(memory_space=pltpu.VMEM)] * 3,
        out_specs=pl.BlockSpec(memory_space=pltpu.VMEM),
        scratch_shapes=[
            pltpu.VMEM((N_DEV, n, h), jnp.bfloat16),
            pltpu.SemaphoreType.DMA((3,)),
            pltpu.SemaphoreType.DMA((3,)),
        ],
        compiler_params=pltpu.CompilerParams(collective_id=0),
    )(xb, route_idx, ws)
